# baseline (speedup 1.0000x reference)
import functools

import jax
import jax.numpy as jnp
from jax import lax
from jax.experimental import pallas as pl
from jax.experimental.pallas import tpu as pltpu
from jax.experimental.pallas import tpu_sc as plsc

VOCAB = 1000000
D_MODEL = 64
SCALE = 8.0

NC, NS = 2, 16
NW = NC * NS
LANES = 16

NSTRIP = VOCAB // 128
REM_BASE = NSTRIP * 128
TSROWS = VOCAB // 2
P1_NBUF = 4
P1_MAXK = (NSTRIP + NW - 1) // NW
P1_NT = (P1_MAXK + P1_NBUF - 1) // P1_NBUF

SEQ, TOK = 4096, 200
SBLK = 128
P2_NBUF = 2

_mesh = plsc.VectorSubcoreMesh(
    core_axis_name="c", subcore_axis_name="s", num_cores=NC, num_subcores=NS
)
_tc_tiled = pltpu.CompilerParams(
    use_tc_tiling_on_sc=True, needs_layout_passes=False
)


def _wid():
    return lax.axis_index("s") * NC + lax.axis_index("c")


def _transpose_strip(in_v, ob_v, nrow, riota, ncol16):
    def row(r, carry):
        c0 = jnp.full((LANES,), 2 * r, jnp.int32)
        c1 = c0 + 1
        for j in range(4):
            if j >= ncol16:
                continue
            v0 = plsc.load_gather(in_v, [riota[j], c0]) * SCALE
            ob_v[r, pl.ds(16 * j, LANES)] = v0
            v1 = plsc.load_gather(in_v, [riota[j], c1]) * SCALE
            ob_v[r, pl.ds(64 + 16 * j, LANES)] = v1
        return carry

    lax.fori_loop(0, nrow, row, 0)


@functools.partial(
    pl.kernel,
    out_type=jax.ShapeDtypeStruct((TSROWS, 128), jnp.float32),
    mesh=_mesh,
    scratch_types=(
        [pltpu.VMEM((64, 128), jnp.float32) for _ in range(2 * P1_NBUF)]
        + [pltpu.SemaphoreType.DMA for _ in range(2 * P1_NBUF)]
    ),
    compiler_params=_tc_tiled,
)
def _repack_table(tt_hbm, tail_hbm, ts_hbm, *rest):
    ins = rest[:P1_NBUF]
    obs = rest[P1_NBUF : 2 * P1_NBUF]
    isem = rest[2 * P1_NBUF : 3 * P1_NBUF]
    osem = rest[3 * P1_NBUF :]

    w = _wid()
    nk = (NSTRIP - w + NW - 1) // NW
    riota = [lax.iota(jnp.int32, LANES) + 16 * j for j in range(4)]

    def strip_of(k):
        return w + k * NW

    def gather_in(b, k):
        c = strip_of(k)
        pltpu.async_copy(
            tt_hbm.at[:, pl.ds(c * 128, 128)], ins[b], isem[b]
        )

    for b in range(P1_NBUF):
        @pl.when(b < nk)
        def _prime(b=b):
            gather_in(b, b)

    def step(t, carry):
        for b in range(P1_NBUF):
            k = t * P1_NBUF + b

            @pl.when(k < nk)
            def _work(b=b, k=k):
                c = strip_of(k)
                pltpu.make_async_copy(
                    tt_hbm.at[:, pl.ds(c * 128, 128)], ins[b], isem[b]
                ).wait()

                @pl.when(k >= P1_NBUF)
                def _free_out():
                    pltpu.make_async_copy(
                        obs[b], ts_hbm.at[pl.ds(0, 64)], osem[b]
                    ).wait()

                _transpose_strip(ins[b], obs[b], 64, riota, 4)
                pltpu.async_copy(obs[b], ts_hbm.at[pl.ds(c * 64, 64)], osem[b])

                @pl.when(k + P1_NBUF < nk)
                def _refill():
                    gather_in(b, k + P1_NBUF)

        return carry

    lax.fori_loop(0, P1_NT, step, 0)

    for b in range(P1_NBUF):
        @pl.when(b < nk)
        def _drain(b=b):
            pltpu.make_async_copy(
                obs[b], ts_hbm.at[pl.ds(0, 64)], osem[b]
            ).wait()

    @pl.when(w == NW - 1)
    def _tail():
        pltpu.sync_copy(tail_hbm, obs[0].at[pl.ds(0, 32)])
        pltpu.sync_copy(
            obs[0].at[pl.ds(0, 32)], ts_hbm.at[pl.ds(REM_BASE // 2, 32)]
        )


@functools.partial(
    pl.kernel,
    out_type=jax.ShapeDtypeStruct((TOK, D_MODEL, SEQ), jnp.float32),
    mesh=_mesh,
    scratch_types=(
        [pltpu.VMEM((TOK, SBLK), jnp.int32)]
        + [pltpu.VMEM((SBLK, 128), jnp.float32) for _ in range(P2_NBUF)]
        + [pltpu.VMEM((D_MODEL, SBLK), jnp.float32) for _ in range(P2_NBUF)]
        + [pltpu.VMEM((SBLK,), jnp.int32) for _ in range(P2_NBUF)]
        + [pltpu.VMEM((SBLK,), jnp.int32) for _ in range(P2_NBUF)]
        + [pltpu.SemaphoreType.DMA for _ in range(2 * P2_NBUF)]
    ),
    compiler_params=_tc_tiled,
)
def _gather_emb(tokt_hbm, ts_hbm, out_hbm, idxslab, *rest):
    bufs = rest[:P2_NBUF]
    obs = rest[P2_NBUF : 2 * P2_NBUF]
    rowv = rest[2 * P2_NBUF : 3 * P2_NBUF]
    parv = rest[3 * P2_NBUF : 4 * P2_NBUF]
    gsem = rest[4 * P2_NBUF : 5 * P2_NBUF]
    osem = rest[5 * P2_NBUF :]

    w = _wid()
    s0 = w * SBLK

    pltpu.sync_copy(tokt_hbm.at[:, pl.ds(s0, SBLK)], idxslab)

    def prep_idx(b, t):
        for g in range(SBLK // LANES):
            tok = idxslab[t, pl.ds(g * LANES, LANES)]
            rowv[b][pl.ds(g * LANES, LANES)] = lax.shift_right_logical(tok, 1)
            parv[b][pl.ds(g * LANES, LANES)] = lax.shift_left(
                lax.bitwise_and(tok, 1), 6
            )

    def gather_start(b):
        pltpu.async_copy(ts_hbm.at[rowv[b]], bufs[b], gsem[b])

    for b in range(P2_NBUF):
        prep_idx(b, b)
        gather_start(b)

    riota = [lax.iota(jnp.int32, LANES) + 16 * g for g in range(SBLK // LANES)]

    def extract(b):
        for g in range(SBLK // LANES):
            par = parv[b][pl.ds(g * LANES, LANES)]

            def drow(d, carry):
                v = plsc.load_gather(bufs[b], [riota[g], par + d])
                obs[b][d, pl.ds(g * LANES, LANES)] = v
                return carry

            lax.fori_loop(0, D_MODEL, drow, 0)

    def step(t2, carry):
        for b in range(P2_NBUF):
            t = t2 * P2_NBUF + b
            pltpu.make_async_copy(ts_hbm.at[rowv[b]], bufs[b], gsem[b]).wait()

            @pl.when(t >= P2_NBUF)
            def _free_out(b=b):
                pltpu.make_async_copy(
                    obs[b], out_hbm.at[0, :, pl.ds(s0, SBLK)], osem[b]
                ).wait()

            extract(b)
            pltpu.async_copy(
                obs[b], out_hbm.at[t, :, pl.ds(s0, SBLK)], osem[b]
            )

            @pl.when(t + P2_NBUF < TOK)
            def _next(b=b, t=t):
                prep_idx(b, t + P2_NBUF)
                gather_start(b)

        return carry

    lax.fori_loop(0, TOK // P2_NBUF, step, 0)

    for b in range(P2_NBUF):
        pltpu.make_async_copy(
            obs[b], out_hbm.at[0, :, pl.ds(s0, SBLK)], osem[b]
        ).wait()


def kernel(tokens, table):
    tail = (table[REM_BASE:] * SCALE).reshape(32, 128)
    ts = _repack_table(table.T, tail)
    out3 = _gather_emb(tokens.astype(jnp.int32).T, ts)
    return out3.transpose(2, 0, 1)


# speedup vs baseline: 1.9478x; 1.9478x over previous
import functools

import jax
import jax.numpy as jnp
from jax import lax
from jax.experimental import pallas as pl
from jax.experimental.pallas import tpu as pltpu
from jax.experimental.pallas import tpu_sc as plsc

VOCAB = 1000000
D_MODEL = 64
SCALE = 8.0

NC, NS = 2, 16
NW = NC * NS
LANES = 16

NSTRIP = VOCAB // 128
REM_BASE = NSTRIP * 128
TSROWS = VOCAB // 2
P1_NBUF = 4
P1_MAXK = (NSTRIP + NW - 1) // NW
P1_NT = (P1_MAXK + P1_NBUF - 1) // P1_NBUF

SEQ, TOK = 4096, 200
SBLK = 128
P2_NBUF = 2

_mesh = plsc.VectorSubcoreMesh(
    core_axis_name="c", subcore_axis_name="s", num_cores=NC, num_subcores=NS
)
_tc_tiled = pltpu.CompilerParams(
    use_tc_tiling_on_sc=True, needs_layout_passes=False
)


def _wid():
    return lax.axis_index("s") * NC + lax.axis_index("c")


def _transpose_strip(in_v, ob_v, nrow, riota, ncol16):
    @plsc.parallel_loop(0, nrow, unroll=4)
    def row(r):
        c0 = jnp.full((LANES,), 2 * r, jnp.int32)
        c1 = c0 + 1
        for j in range(4):
            if j >= ncol16:
                continue
            v0 = plsc.load_gather(in_v, [riota[j], c0]) * SCALE
            ob_v[r, pl.ds(16 * j, LANES)] = v0
            v1 = plsc.load_gather(in_v, [riota[j], c1]) * SCALE
            ob_v[r, pl.ds(64 + 16 * j, LANES)] = v1


@functools.partial(
    pl.kernel,
    out_type=jax.ShapeDtypeStruct((TSROWS, 128), jnp.float32),
    mesh=_mesh,
    scratch_types=(
        [pltpu.VMEM((64, 128), jnp.float32) for _ in range(2 * P1_NBUF)]
        + [pltpu.SemaphoreType.DMA for _ in range(2 * P1_NBUF)]
    ),
    compiler_params=_tc_tiled,
)
def _repack_table(tt_hbm, tail_hbm, ts_hbm, *rest):
    ins = rest[:P1_NBUF]
    obs = rest[P1_NBUF : 2 * P1_NBUF]
    isem = rest[2 * P1_NBUF : 3 * P1_NBUF]
    osem = rest[3 * P1_NBUF :]

    w = _wid()
    nk = (NSTRIP - w + NW - 1) // NW
    riota = [lax.iota(jnp.int32, LANES) + 16 * j for j in range(4)]

    def strip_of(k):
        return w + k * NW

    def gather_in(b, k):
        c = strip_of(k)
        pltpu.async_copy(
            tt_hbm.at[:, pl.ds(c * 128, 128)], ins[b], isem[b]
        )

    for b in range(P1_NBUF):
        @pl.when(b < nk)
        def _prime(b=b):
            gather_in(b, b)

    def step(t, carry):
        for b in range(P1_NBUF):
            k = t * P1_NBUF + b

            @pl.when(k < nk)
            def _work(b=b, k=k):
                c = strip_of(k)
                pltpu.make_async_copy(
                    tt_hbm.at[:, pl.ds(c * 128, 128)], ins[b], isem[b]
                ).wait()

                @pl.when(k >= P1_NBUF)
                def _free_out():
                    pltpu.make_async_copy(
                        obs[b], ts_hbm.at[pl.ds(0, 64)], osem[b]
                    ).wait()

                _transpose_strip(ins[b], obs[b], 64, riota, 4)
                pltpu.async_copy(obs[b], ts_hbm.at[pl.ds(c * 64, 64)], osem[b])

                @pl.when(k + P1_NBUF < nk)
                def _refill():
                    gather_in(b, k + P1_NBUF)

        return carry

    lax.fori_loop(0, P1_NT, step, 0)

    for b in range(P1_NBUF):
        @pl.when(b < nk)
        def _drain(b=b):
            pltpu.make_async_copy(
                obs[b], ts_hbm.at[pl.ds(0, 64)], osem[b]
            ).wait()

    @pl.when(w == NW - 1)
    def _tail():
        pltpu.sync_copy(tail_hbm, obs[0].at[pl.ds(0, 32)])
        pltpu.sync_copy(
            obs[0].at[pl.ds(0, 32)], ts_hbm.at[pl.ds(REM_BASE // 2, 32)]
        )


@functools.partial(
    pl.kernel,
    out_type=jax.ShapeDtypeStruct((TOK, D_MODEL, SEQ), jnp.float32),
    mesh=_mesh,
    scratch_types=(
        [pltpu.VMEM((TOK, SBLK), jnp.int32)]
        + [pltpu.VMEM((SBLK, 128), jnp.float32) for _ in range(P2_NBUF)]
        + [pltpu.VMEM((D_MODEL, SBLK), jnp.float32) for _ in range(P2_NBUF)]
        + [pltpu.VMEM((SBLK,), jnp.int32) for _ in range(P2_NBUF)]
        + [pltpu.VMEM((SBLK,), jnp.int32) for _ in range(P2_NBUF)]
        + [pltpu.SemaphoreType.DMA for _ in range(2 * P2_NBUF)]
    ),
    compiler_params=_tc_tiled,
)
def _gather_emb(tokt_hbm, ts_hbm, out_hbm, idxslab, *rest):
    bufs = rest[:P2_NBUF]
    obs = rest[P2_NBUF : 2 * P2_NBUF]
    rowv = rest[2 * P2_NBUF : 3 * P2_NBUF]
    parv = rest[3 * P2_NBUF : 4 * P2_NBUF]
    gsem = rest[4 * P2_NBUF : 5 * P2_NBUF]
    osem = rest[5 * P2_NBUF :]

    w = _wid()
    s0 = w * SBLK

    pltpu.sync_copy(tokt_hbm.at[:, pl.ds(s0, SBLK)], idxslab)

    def prep_idx(b, t):
        for g in range(SBLK // LANES):
            tok = idxslab[t, pl.ds(g * LANES, LANES)]
            rowv[b][pl.ds(g * LANES, LANES)] = lax.shift_right_logical(tok, 1)
            parv[b][pl.ds(g * LANES, LANES)] = lax.shift_left(
                lax.bitwise_and(tok, 1), 6
            )

    def gather_start(b):
        pltpu.async_copy(ts_hbm.at[rowv[b]], bufs[b], gsem[b])

    for b in range(P2_NBUF):
        prep_idx(b, b)
        gather_start(b)

    riota = [lax.iota(jnp.int32, LANES) + 16 * g for g in range(SBLK // LANES)]

    def extract(b):
        for g in range(SBLK // LANES):
            par = parv[b][pl.ds(g * LANES, LANES)]

            @plsc.parallel_loop(0, D_MODEL, unroll=4)
            def drow(d, g=g, par=par):
                v = plsc.load_gather(bufs[b], [riota[g], par + d])
                obs[b][d, pl.ds(g * LANES, LANES)] = v

    def step(t2, carry):
        for b in range(P2_NBUF):
            t = t2 * P2_NBUF + b
            pltpu.make_async_copy(ts_hbm.at[rowv[b]], bufs[b], gsem[b]).wait()

            @pl.when(t >= P2_NBUF)
            def _free_out(b=b):
                pltpu.make_async_copy(
                    obs[b], out_hbm.at[0, :, pl.ds(s0, SBLK)], osem[b]
                ).wait()

            extract(b)
            pltpu.async_copy(
                obs[b], out_hbm.at[t, :, pl.ds(s0, SBLK)], osem[b]
            )

            @pl.when(t + P2_NBUF < TOK)
            def _next(b=b, t=t):
                prep_idx(b, t + P2_NBUF)
                gather_start(b)

        return carry

    lax.fori_loop(0, TOK // P2_NBUF, step, 0)

    for b in range(P2_NBUF):
        pltpu.make_async_copy(
            obs[b], out_hbm.at[0, :, pl.ds(s0, SBLK)], osem[b]
        ).wait()


def kernel(tokens, table):
    tail = (table[REM_BASE:] * SCALE).reshape(32, 128)
    ts = _repack_table(table.T, tail)
    out3 = _gather_emb(tokens.astype(jnp.int32).T, ts)
    return out3.transpose(2, 0, 1)


# speedup vs baseline: 1.9509x; 1.0016x over previous
import functools

import jax
import jax.numpy as jnp
from jax import lax
from jax.experimental import pallas as pl
from jax.experimental.pallas import tpu as pltpu
from jax.experimental.pallas import tpu_sc as plsc

VOCAB = 1000000
D_MODEL = 64
SCALE = 8.0

NC, NS = 2, 16
NW = NC * NS
LANES = 16

NSTRIP = VOCAB // 128
REM_BASE = NSTRIP * 128
TSROWS = VOCAB // 2
P1_NBUF = 4
P1_MAXK = (NSTRIP + NW - 1) // NW
P1_NT = (P1_MAXK + P1_NBUF - 1) // P1_NBUF

SEQ, TOK = 4096, 200
SBLK = 128
P2_NBUF = 2

_mesh = plsc.VectorSubcoreMesh(
    core_axis_name="c", subcore_axis_name="s", num_cores=NC, num_subcores=NS
)
_tc_tiled = pltpu.CompilerParams(
    use_tc_tiling_on_sc=True, needs_layout_passes=False
)


def _wid():
    return lax.axis_index("s") * NC + lax.axis_index("c")


def _transpose_strip(in_v, ob_v, nrow, riota128):
    in1 = in_v.reshape(1, 64 * 128)
    zero = jnp.zeros((LANES,), jnp.int32)

    @plsc.parallel_loop(0, nrow, unroll=8)
    def row(r):
        for h in range(2):
            for j in range(4):
                v = plsc.load_gather(in1, [zero, riota128[j] + (2 * r + h)])
                ob_v[r, pl.ds(h * 64 + 16 * j, LANES)] = v * SCALE


@functools.partial(
    pl.kernel,
    out_type=jax.ShapeDtypeStruct((TSROWS, 128), jnp.float32),
    mesh=_mesh,
    scratch_types=(
        [pltpu.VMEM((64, 128), jnp.float32) for _ in range(2 * P1_NBUF)]
        + [pltpu.SemaphoreType.DMA for _ in range(2 * P1_NBUF)]
    ),
    compiler_params=_tc_tiled,
)
def _repack_table(tt_hbm, tail_hbm, ts_hbm, *rest):
    ins = rest[:P1_NBUF]
    obs = rest[P1_NBUF : 2 * P1_NBUF]
    isem = rest[2 * P1_NBUF : 3 * P1_NBUF]
    osem = rest[3 * P1_NBUF :]

    w = _wid()
    nk = (NSTRIP - w + NW - 1) // NW
    riota128 = [(lax.iota(jnp.int32, LANES) + 16 * j) * 128 for j in range(4)]

    def strip_of(k):
        return w + k * NW

    def gather_in(b, k):
        c = strip_of(k)
        pltpu.async_copy(
            tt_hbm.at[:, pl.ds(c * 128, 128)], ins[b], isem[b]
        )

    for b in range(P1_NBUF):
        @pl.when(b < nk)
        def _prime(b=b):
            gather_in(b, b)

    def step(t, carry):
        for b in range(P1_NBUF):
            k = t * P1_NBUF + b

            @pl.when(k < nk)
            def _work(b=b, k=k):
                c = strip_of(k)
                pltpu.make_async_copy(
                    tt_hbm.at[:, pl.ds(c * 128, 128)], ins[b], isem[b]
                ).wait()

                @pl.when(k >= P1_NBUF)
                def _free_out():
                    pltpu.make_async_copy(
                        obs[b], ts_hbm.at[pl.ds(0, 64)], osem[b]
                    ).wait()

                _transpose_strip(ins[b], obs[b], 64, riota128)
                pltpu.async_copy(obs[b], ts_hbm.at[pl.ds(c * 64, 64)], osem[b])

                @pl.when(k + P1_NBUF < nk)
                def _refill():
                    gather_in(b, k + P1_NBUF)

        return carry

    lax.fori_loop(0, P1_NT, step, 0)

    for b in range(P1_NBUF):
        @pl.when(b < nk)
        def _drain(b=b):
            pltpu.make_async_copy(
                obs[b], ts_hbm.at[pl.ds(0, 64)], osem[b]
            ).wait()

    @pl.when(w == NW - 1)
    def _tail():
        pltpu.sync_copy(tail_hbm, obs[0].at[pl.ds(0, 32)])
        pltpu.sync_copy(
            obs[0].at[pl.ds(0, 32)], ts_hbm.at[pl.ds(REM_BASE // 2, 32)]
        )


@functools.partial(
    pl.kernel,
    out_type=jax.ShapeDtypeStruct((TOK, D_MODEL, SEQ), jnp.float32),
    mesh=_mesh,
    scratch_types=(
        [pltpu.VMEM((TOK, SBLK), jnp.int32)]
        + [pltpu.VMEM((SBLK, 128), jnp.float32) for _ in range(P2_NBUF)]
        + [pltpu.VMEM((D_MODEL, SBLK), jnp.float32) for _ in range(P2_NBUF)]
        + [pltpu.VMEM((SBLK,), jnp.int32) for _ in range(P2_NBUF)]
        + [pltpu.VMEM((SBLK,), jnp.int32) for _ in range(P2_NBUF)]
        + [pltpu.SemaphoreType.DMA for _ in range(2 * P2_NBUF)]
    ),
    compiler_params=_tc_tiled,
)
def _gather_emb(tokt_hbm, ts_hbm, out_hbm, idxslab, *rest):
    bufs = rest[:P2_NBUF]
    obs = rest[P2_NBUF : 2 * P2_NBUF]
    rowv = rest[2 * P2_NBUF : 3 * P2_NBUF]
    parv = rest[3 * P2_NBUF : 4 * P2_NBUF]
    gsem = rest[4 * P2_NBUF : 5 * P2_NBUF]
    osem = rest[5 * P2_NBUF :]

    w = _wid()
    s0 = w * SBLK

    pltpu.sync_copy(tokt_hbm.at[:, pl.ds(s0, SBLK)], idxslab)

    def prep_idx(b, t):
        for g in range(SBLK // LANES):
            tok = idxslab[t, pl.ds(g * LANES, LANES)]
            rowv[b][pl.ds(g * LANES, LANES)] = lax.shift_right_logical(tok, 1)
            parv[b][pl.ds(g * LANES, LANES)] = lax.shift_left(
                lax.bitwise_and(tok, 1), 6
            )

    def gather_start(b):
        pltpu.async_copy(ts_hbm.at[rowv[b]], bufs[b], gsem[b])

    for b in range(P2_NBUF):
        prep_idx(b, b)
        gather_start(b)

    riota128 = [
        (lax.iota(jnp.int32, LANES) + 16 * g) * 128 for g in range(SBLK // LANES)
    ]

    def extract(b):
        b1 = bufs[b].reshape(1, SBLK * 128)
        zero = jnp.zeros((LANES,), jnp.int32)
        for g in range(SBLK // LANES):
            par128 = riota128[g] + parv[b][pl.ds(g * LANES, LANES)]

            @plsc.parallel_loop(0, D_MODEL, unroll=8)
            def drow(d, g=g, par128=par128):
                v = plsc.load_gather(b1, [zero, par128 + d])
                obs[b][d, pl.ds(g * LANES, LANES)] = v

    def step(t2, carry):
        for b in range(P2_NBUF):
            t = t2 * P2_NBUF + b
            pltpu.make_async_copy(ts_hbm.at[rowv[b]], bufs[b], gsem[b]).wait()

            @pl.when(t >= P2_NBUF)
            def _free_out(b=b):
                pltpu.make_async_copy(
                    obs[b], out_hbm.at[0, :, pl.ds(s0, SBLK)], osem[b]
                ).wait()

            extract(b)
            pltpu.async_copy(
                obs[b], out_hbm.at[t, :, pl.ds(s0, SBLK)], osem[b]
            )

            @pl.when(t + P2_NBUF < TOK)
            def _next(b=b, t=t):
                prep_idx(b, t + P2_NBUF)
                gather_start(b)

        return carry

    lax.fori_loop(0, TOK // P2_NBUF, step, 0)

    for b in range(P2_NBUF):
        pltpu.make_async_copy(
            obs[b], out_hbm.at[0, :, pl.ds(s0, SBLK)], osem[b]
        ).wait()


def kernel(tokens, table):
    tail = (table[REM_BASE:] * SCALE).reshape(32, 128)
    ts = _repack_table(table.T, tail)
    out3 = _gather_emb(tokens.astype(jnp.int32).T, ts)
    return out3.transpose(2, 0, 1)
